# Initial kernel scaffold; baseline (speedup 1.0000x reference)
import jax
import jax.numpy as jnp
from jax.experimental import pallas as pl


def kernel(logits):
    raise NotImplementedError("write your pallas kernel here")


# speedup vs baseline: 15.1638x; 15.1638x over previous
import functools

import jax
import jax.numpy as jnp
from jax import lax
from jax.experimental import pallas as pl
from jax.experimental.pallas import tpu as pltpu
from jax.experimental.pallas import tpu_sc as plsc

TRUNC_R = 0.85
NEG_FILL = -70.0

_B, _V, _S = 16, 1024, 2048
_L = 16

_info = plsc.get_sparse_core_info()
_NC, _NS = _info.num_cores, _info.num_subcores
_NW = _NC * _NS
_TCH = _S // _L
_NTASK = (_B * _TCH) // _NW


def _enc(v):
    u = plsc.bitcast(v, jnp.uint32)
    s = u >> jnp.uint32(31)
    mask = s * jnp.uint32(0x7FFFFFFF) + jnp.uint32(0x80000000)
    return u ^ mask


def _dec(e):
    s = e >> jnp.uint32(31)
    mask = jnp.uint32(0xFFFFFFFF) - s * jnp.uint32(0x7FFFFFFF)
    return plsc.bitcast(e ^ mask, jnp.float32)


def _body(x_hbm, out_hbm, in_v, out_v):
    wid = lax.axis_index("s") * _NC + lax.axis_index("c")

    def task(k, carry):
        t = wid * _NTASK + k
        b = t // _TCH
        t0 = (t % _TCH) * _L
        pltpu.sync_copy(x_hbm.at[b, :, pl.ds(t0, _L)], in_v)

        def p1(i, m):
            return jnp.maximum(m, in_v[i, :])

        m = lax.fori_loop(0, _V, p1, jnp.full((_L,), -jnp.inf, jnp.float32))

        def mass_gt(thr):
            def mb(i, acc):
                v = in_v[i, :]
                return acc + jnp.where(_enc(v) > thr, jnp.exp(v), 0.0)

            return lax.fori_loop(0, _V, mb, jnp.zeros((_L,), jnp.float32))

        def common(_):
            return _enc(m), jnp.zeros((_L,), jnp.float32)

        def rare(_):
            def sb(_j, c):
                lo, hi = c
                mid = lo + ((hi - lo) >> jnp.uint32(1))
                pred = mass_gt(mid) < TRUNC_R
                return (jnp.where(pred, lo, mid + jnp.uint32(1)),
                        jnp.where(pred, mid, hi))

            lo0 = jnp.zeros((_L,), jnp.uint32)
            hi0 = jnp.full((_L,), 0xFFFFFFFF, jnp.uint32)
            _, thr = lax.fori_loop(0, 32, sb, (lo0, hi0))
            return thr, mass_gt(thr)

        any_rare = plsc.all_reduce_population_count(jnp.exp(m) < TRUNC_R)[0] > 0
        T, A = lax.cond(any_rare, rare, common, None)
        e_t = jnp.exp(_dec(T))

        def p2(i, s):
            v = in_v[i, :]
            ev = _enc(v)
            eq = ev == T
            keep = (ev > T) | (eq & (s < TRUNC_R))
            out_v[i, :] = jnp.where(keep, v, NEG_FILL)
            return s + jnp.where(eq, e_t, 0.0)

        lax.fori_loop(0, _V, p2, A)
        pltpu.sync_copy(out_v, out_hbm.at[b, :, pl.ds(t0, _L)])
        return carry

    lax.fori_loop(0, _NTASK, task, 0)


_sc_call = functools.partial(
    pl.kernel,
    out_type=jax.ShapeDtypeStruct((_B, _V, _S), jnp.float32),
    mesh=plsc.VectorSubcoreMesh(core_axis_name="c", subcore_axis_name="s"),
    scratch_types=[
        pltpu.VMEM((_V, _L), jnp.float32),
        pltpu.VMEM((_V, _L), jnp.float32),
    ],
    compiler_params=pltpu.CompilerParams(
        use_tc_tiling_on_sc=False, needs_layout_passes=False
    ),
)(_body)


@jax.jit
def kernel(logits):
    return _sc_call(logits)


# speedup vs baseline: 20.4236x; 1.3469x over previous
import functools

import jax
import jax.numpy as jnp
from jax import lax
from jax.experimental import pallas as pl
from jax.experimental.pallas import tpu as pltpu
from jax.experimental.pallas import tpu_sc as plsc

TRUNC_R = 0.85
NEG_FILL = -70.0

_B, _V, _S = 16, 1024, 2048
_L = 16

_info = plsc.get_sparse_core_info()
_NC, _NS = _info.num_cores, _info.num_subcores
_NW = _NC * _NS
_TCH = _S // _L
_NTASK = (_B * _TCH) // _NW


def _enc(v):
    u = plsc.bitcast(v, jnp.uint32)
    s = u >> jnp.uint32(31)
    mask = s * jnp.uint32(0x7FFFFFFF) + jnp.uint32(0x80000000)
    return u ^ mask


def _dec(e):
    s = e >> jnp.uint32(31)
    mask = jnp.uint32(0xFFFFFFFF) - s * jnp.uint32(0x7FFFFFFF)
    return plsc.bitcast(e ^ mask, jnp.float32)


def _body(x_hbm, out_hbm, in_v, out_v):
    wid = lax.axis_index("s") * _NC + lax.axis_index("c")

    def task(k, carry):
        t = wid * _NTASK + k
        b = t // _TCH
        t0 = (t % _TCH) * _L
        pltpu.sync_copy(x_hbm.at[b, :, pl.ds(t0, _L)], in_v)

        def p1(i, ms):
            base = i * 8
            return tuple(
                jnp.maximum(ms[j], in_v[base + j, :]) for j in range(8)
            )

        ms = lax.fori_loop(
            0, _V // 8, p1,
            tuple(jnp.full((_L,), -jnp.inf, jnp.float32) for _ in range(8)),
        )
        m = jnp.maximum(
            jnp.maximum(jnp.maximum(ms[0], ms[1]), jnp.maximum(ms[2], ms[3])),
            jnp.maximum(jnp.maximum(ms[4], ms[5]), jnp.maximum(ms[6], ms[7])),
        )

        def mass_gt(thr):
            def mb(i, accs):
                base = i * 4
                return tuple(
                    accs[j]
                    + jnp.where(
                        _enc(in_v[base + j, :]) > thr,
                        jnp.exp(in_v[base + j, :]),
                        0.0,
                    )
                    for j in range(4)
                )

            accs = lax.fori_loop(
                0, _V // 4, mb,
                tuple(jnp.zeros((_L,), jnp.float32) for _ in range(4)),
            )
            return (accs[0] + accs[1]) + (accs[2] + accs[3])

        def common(_):
            return _enc(m), jnp.zeros((_L,), jnp.float32)

        def rare(_):
            def sb(_j, c):
                lo, hi = c
                mid = lo + ((hi - lo) >> jnp.uint32(1))
                pred = mass_gt(mid) < TRUNC_R
                return (jnp.where(pred, lo, mid + jnp.uint32(1)),
                        jnp.where(pred, mid, hi))

            lo0 = jnp.zeros((_L,), jnp.uint32)
            hi0 = jnp.full((_L,), 0xFFFFFFFF, jnp.uint32)
            _, thr = lax.fori_loop(0, 32, sb, (lo0, hi0))
            return thr, mass_gt(thr)

        any_rare = plsc.all_reduce_population_count(jnp.exp(m) < TRUNC_R)[0] > 0
        T, A = lax.cond(any_rare, rare, common, None)
        e_t = jnp.exp(_dec(T))

        def p2(i, s):
            base = i * 8
            vs = [in_v[base + j, :] for j in range(8)]
            evs = [_enc(v) for v in vs]
            eqs = [ev == T for ev in evs]
            adds = [jnp.where(eq, e_t, 0.0) for eq in eqs]
            t01 = adds[0] + adds[1]
            t23 = adds[2] + adds[3]
            t45 = adds[4] + adds[5]
            t67 = adds[6] + adds[7]
            p4 = t01 + t23
            p6 = p4 + t45
            pr = (s, s + adds[0], s + t01, s + (t01 + adds[2]),
                  s + p4, s + (p4 + adds[4]), s + p6, s + (p6 + adds[6]))
            for j in range(8):
                keep = (evs[j] > T) | (eqs[j] & (pr[j] < TRUNC_R))
                out_v[base + j, :] = jnp.where(keep, vs[j], NEG_FILL)
            return s + (p4 + (t45 + t67))

        lax.fori_loop(0, _V // 8, p2, A)
        pltpu.sync_copy(out_v, out_hbm.at[b, :, pl.ds(t0, _L)])
        return carry

    lax.fori_loop(0, _NTASK, task, 0)


_sc_call = functools.partial(
    pl.kernel,
    out_type=jax.ShapeDtypeStruct((_B, _V, _S), jnp.float32),
    mesh=plsc.VectorSubcoreMesh(core_axis_name="c", subcore_axis_name="s"),
    scratch_types=[
        pltpu.VMEM((_V, _L), jnp.float32),
        pltpu.VMEM((_V, _L), jnp.float32),
    ],
    compiler_params=pltpu.CompilerParams(
        use_tc_tiling_on_sc=False, needs_layout_passes=False
    ),
)(_body)


@jax.jit
def kernel(logits):
    return _sc_call(logits)


# speedup vs baseline: 28.5578x; 1.3983x over previous
import functools

import jax
import jax.numpy as jnp
from jax import lax
from jax.experimental import pallas as pl
from jax.experimental.pallas import tpu as pltpu
from jax.experimental.pallas import tpu_sc as plsc

TRUNC_R = 0.85
NEG_FILL = -70.0

_B, _V, _S = 16, 1024, 2048
_L = 16

_info = plsc.get_sparse_core_info()
_NC, _NS = _info.num_cores, _info.num_subcores
_NW = _NC * _NS
_TCH = _S // _L
_NTASK = (_B * _TCH) // _NW


def _enc(v):
    u = plsc.bitcast(v, jnp.uint32)
    s = u >> jnp.uint32(31)
    mask = s * jnp.uint32(0x7FFFFFFF) + jnp.uint32(0x80000000)
    return u ^ mask


def _dec(e):
    s = e >> jnp.uint32(31)
    mask = jnp.uint32(0xFFFFFFFF) - s * jnp.uint32(0x7FFFFFFF)
    return plsc.bitcast(e ^ mask, jnp.float32)


def _in_slice(x_hbm, t):
    b = t // _TCH
    t0 = (t % _TCH) * _L
    return x_hbm.at[b, :, pl.ds(t0, _L)]


def _body(x_hbm, out_hbm, in_v0, in_v1, out_v0, out_v1, is0, is1, os0, os1):
    wid = lax.axis_index("s") * _NC + lax.axis_index("c")
    tbase = wid * _NTASK
    ins = ((in_v0, is0), (in_v1, is1))
    outs = ((out_v0, os0), (out_v1, os1))

    pltpu.async_copy(_in_slice(x_hbm, tbase), in_v0, is0)

    def pair(i, carry):
        for u in (0, 1):
            in_v, isem = ins[u]
            out_v, osem = outs[u]
            k = 2 * i + u
            t = tbase + k
            pltpu.make_async_copy(_in_slice(x_hbm, t), in_v, isem).wait()
            nin_v, nisem = ins[1 - u]

            @pl.when(k + 1 < _NTASK)
            def _():
                pltpu.async_copy(_in_slice(x_hbm, t + 1), nin_v, nisem)

            @pl.when(i >= 1)
            def _():
                pltpu.make_async_copy(
                    out_v, _in_slice(out_hbm, t - 2), osem
                ).wait()

            _tile(x_hbm, out_hbm, in_v, out_v, t)
            pltpu.async_copy(out_v, _in_slice(out_hbm, t), osem)
        return carry

    lax.fori_loop(0, _NTASK // 2, pair, 0)
    pltpu.make_async_copy(
        out_v0, _in_slice(out_hbm, tbase + _NTASK - 2), os0
    ).wait()
    pltpu.make_async_copy(
        out_v1, _in_slice(out_hbm, tbase + _NTASK - 1), os1
    ).wait()


def _tile(x_hbm, out_hbm, in_v, out_v, t):
    if True:

        def p1(i, ms):
            base = i * 8
            return tuple(
                jnp.maximum(ms[j], in_v[base + j, :]) for j in range(8)
            )

        ms = lax.fori_loop(
            0, _V // 8, p1,
            tuple(jnp.full((_L,), -jnp.inf, jnp.float32) for _ in range(8)),
        )
        m = jnp.maximum(
            jnp.maximum(jnp.maximum(ms[0], ms[1]), jnp.maximum(ms[2], ms[3])),
            jnp.maximum(jnp.maximum(ms[4], ms[5]), jnp.maximum(ms[6], ms[7])),
        )

        def mass_gt(thr):
            def mb(i, accs):
                base = i * 4
                return tuple(
                    accs[j]
                    + jnp.where(
                        _enc(in_v[base + j, :]) > thr,
                        jnp.exp(in_v[base + j, :]),
                        0.0,
                    )
                    for j in range(4)
                )

            accs = lax.fori_loop(
                0, _V // 4, mb,
                tuple(jnp.zeros((_L,), jnp.float32) for _ in range(4)),
            )
            return (accs[0] + accs[1]) + (accs[2] + accs[3])

        def common(_):
            return _enc(m), jnp.zeros((_L,), jnp.float32)

        def rare(_):
            def sb(_j, c):
                lo, hi = c
                mid = lo + ((hi - lo) >> jnp.uint32(1))
                pred = mass_gt(mid) < TRUNC_R
                return (jnp.where(pred, lo, mid + jnp.uint32(1)),
                        jnp.where(pred, mid, hi))

            lo0 = jnp.zeros((_L,), jnp.uint32)
            hi0 = jnp.full((_L,), 0xFFFFFFFF, jnp.uint32)
            _, thr = lax.fori_loop(0, 32, sb, (lo0, hi0))
            return thr, mass_gt(thr)

        any_rare = plsc.all_reduce_population_count(jnp.exp(m) < TRUNC_R)[0] > 0
        T, A = lax.cond(any_rare, rare, common, None)
        e_t = jnp.exp(_dec(T))

        def p2(i, s):
            base = i * 8
            vs = [in_v[base + j, :] for j in range(8)]
            evs = [_enc(v) for v in vs]
            eqs = [ev == T for ev in evs]
            adds = [jnp.where(eq, e_t, 0.0) for eq in eqs]
            t01 = adds[0] + adds[1]
            t23 = adds[2] + adds[3]
            t45 = adds[4] + adds[5]
            t67 = adds[6] + adds[7]
            p4 = t01 + t23
            p6 = p4 + t45
            pr = (s, s + adds[0], s + t01, s + (t01 + adds[2]),
                  s + p4, s + (p4 + adds[4]), s + p6, s + (p6 + adds[6]))
            for j in range(8):
                keep = (evs[j] > T) | (eqs[j] & (pr[j] < TRUNC_R))
                out_v[base + j, :] = jnp.where(keep, vs[j], NEG_FILL)
            return s + (p4 + (t45 + t67))

        lax.fori_loop(0, _V // 8, p2, A)


_sc_call = functools.partial(
    pl.kernel,
    out_type=jax.ShapeDtypeStruct((_B, _V, _S), jnp.float32),
    mesh=plsc.VectorSubcoreMesh(core_axis_name="c", subcore_axis_name="s"),
    scratch_types=[
        pltpu.VMEM((_V, _L), jnp.float32),
        pltpu.VMEM((_V, _L), jnp.float32),
        pltpu.VMEM((_V, _L), jnp.float32),
        pltpu.VMEM((_V, _L), jnp.float32),
        pltpu.SemaphoreType.DMA,
        pltpu.SemaphoreType.DMA,
        pltpu.SemaphoreType.DMA,
        pltpu.SemaphoreType.DMA,
    ],
    compiler_params=pltpu.CompilerParams(
        use_tc_tiling_on_sc=False, needs_layout_passes=False
    ),
)(_body)


@jax.jit
def kernel(logits):
    return _sc_call(logits)


# speedup vs baseline: 72.7292x; 2.5467x over previous
import functools

import jax
import jax.numpy as jnp
from jax import lax
from jax.experimental import pallas as pl
from jax.experimental.pallas import tpu as pltpu
from jax.experimental.pallas import tpu_sc as plsc

TRUNC_R = 0.85
NEG_FILL = -70.0

_B, _V, _S = 16, 1024, 2048
_L = 16
_W = 128
_G = _W // _L
_CH = 256
_NCH = _V // _CH

_info = plsc.get_sparse_core_info()
_NC, _NS = _info.num_cores, _info.num_subcores
_NW = _NC * _NS
_NSLAB = (_B * (_S // _W)) // _NW


def _enc(v):
    u = plsc.bitcast(v, jnp.uint32)
    s = u >> jnp.uint32(31)
    mask = s * jnp.uint32(0x7FFFFFFF) + jnp.uint32(0x80000000)
    return u ^ mask


def _dec(e):
    s = e >> jnp.uint32(31)
    mask = jnp.uint32(0xFFFFFFFF) - s * jnp.uint32(0x7FFFFFFF)
    return plsc.bitcast(e ^ mask, jnp.float32)


def _chunk(ref, b, t0, c):
    return ref.at[b, pl.ds(c * _CH, _CH), pl.ds(t0, _W)]


def _body(x_hbm, out_hbm, b0, b1, b2, is0, is1, is2, os0, os1, os2):
    wid = lax.axis_index("s") * _NC + lax.axis_index("c")
    bufs = (b0, b1, b2)
    isems = (is0, is1, is2)
    osems = (os0, os1, os2)

    def slab(j, carry):
        s = wid * _NSLAB + j
        b = s // (_S // _W)
        t0 = (s % (_S // _W)) * _W

        for c in range(3):
            pltpu.async_copy(_chunk(x_hbm, b, t0, c), bufs[c], isems[c])

        def p1(buf):
            def rows(i, mm):
                m1, m2 = mm
                base = i * 2
                for r in range(2):
                    vs = [buf[base + r, pl.ds(16 * g, 16)] for g in range(_G)]
                    mn = [jnp.minimum(m1[g], vs[g]) for g in range(_G)]
                    m1 = tuple(
                        jnp.maximum(m1[g], vs[g]) for g in range(_G)
                    )
                    m2 = tuple(jnp.maximum(m2[g], mn[g]) for g in range(_G))
                return m1, m2

            return rows

        ninf = tuple(jnp.full((_L,), -jnp.inf, jnp.float32) for _ in range(_G))
        mm = (ninf, ninf)
        for c in range(_NCH):
            pltpu.make_async_copy(
                _chunk(x_hbm, b, t0, c), bufs[c % 3], isems[c % 3]
            ).wait()
            if c + 1 < _NCH:
                pltpu.async_copy(
                    _chunk(x_hbm, b, t0, c + 1), bufs[(c + 1) % 3],
                    isems[(c + 1) % 3],
                )
            mm = lax.fori_loop(0, _CH // 2, p1(bufs[c % 3]), mm)
        m1, m2 = mm

        em = [jnp.exp(m1[g]) for g in range(_G)]
        tie = [m2[g] == m1[g] for g in range(_G)]
        rare = [em[g] < TRUNC_R for g in range(_G)]
        n_slow = sum(
            plsc.all_reduce_population_count(tie[g] | rare[g])[0]
            for g in range(_G)
        )
        n_rare = sum(
            plsc.all_reduce_population_count(rare[g])[0] for g in range(_G)
        )

        def fast(_):
            def pf(buf):
                def rows(i, carry):
                    for g in range(_G):
                        v = buf[i, pl.ds(16 * g, 16)]
                        buf[i, pl.ds(16 * g, 16)] = jnp.where(
                            v == m1[g], v, NEG_FILL
                        )
                    return carry

                return rows

            lax.fori_loop(0, _CH, pf(b0), 0)
            pltpu.async_copy(b0, _chunk(out_hbm, b, t0, 3), os0)
            lax.fori_loop(0, _CH, pf(b2), 0)
            pltpu.async_copy(b2, _chunk(out_hbm, b, t0, 2), os2)
            pltpu.make_async_copy(b0, _chunk(out_hbm, b, t0, 3), os0).wait()
            pltpu.async_copy(_chunk(x_hbm, b, t0, 0), b0, is0)
            lax.fori_loop(0, _CH, pf(b1), 0)
            pltpu.async_copy(b1, _chunk(out_hbm, b, t0, 1), os1)
            pltpu.make_async_copy(_chunk(x_hbm, b, t0, 0), b0, is0).wait()
            lax.fori_loop(0, _CH, pf(b0), 0)
            pltpu.async_copy(b0, _chunk(out_hbm, b, t0, 0), os0)
            pltpu.make_async_copy(b0, _chunk(out_hbm, b, t0, 0), os0).wait()
            pltpu.make_async_copy(b1, _chunk(out_hbm, b, t0, 1), os1).wait()
            pltpu.make_async_copy(b2, _chunk(out_hbm, b, t0, 2), os2).wait()
            return 0

        def slow(_):
            def mass_gt(thr):
                def rows(i, acc):
                    return tuple(
                        acc[g]
                        + jnp.where(
                            _enc(b0[i, pl.ds(16 * g, 16)]) > thr[g],
                            jnp.exp(b0[i, pl.ds(16 * g, 16)]),
                            0.0,
                        )
                        for g in range(_G)
                    )

                acc = tuple(jnp.zeros((_L,), jnp.float32) for _ in range(_G))
                for c in range(_NCH):
                    pltpu.sync_copy(_chunk(x_hbm, b, t0, c), b0)
                    acc = lax.fori_loop(0, _CH, rows, acc)
                return acc

            def common(_):
                return (
                    tuple(_enc(m1[g]) for g in range(_G)),
                    tuple(jnp.zeros((_L,), jnp.float32) for _ in range(_G)),
                )

            def search(_):
                def sb(_i, c):
                    lo, hi = c
                    mid = tuple(
                        lo[g] + ((hi[g] - lo[g]) >> jnp.uint32(1))
                        for g in range(_G)
                    )
                    pred = [mg < TRUNC_R for mg in mass_gt(mid)]
                    return (
                        tuple(
                            jnp.where(pred[g], lo[g], mid[g] + jnp.uint32(1))
                            for g in range(_G)
                        ),
                        tuple(
                            jnp.where(pred[g], mid[g], hi[g])
                            for g in range(_G)
                        ),
                    )

                lo0 = tuple(jnp.zeros((_L,), jnp.uint32) for _ in range(_G))
                hi0 = tuple(
                    jnp.full((_L,), 0xFFFFFFFF, jnp.uint32) for _ in range(_G)
                )
                _, thr = lax.fori_loop(0, 32, sb, (lo0, hi0))
                return thr, mass_gt(thr)

            T, A = lax.cond(n_rare > 0, search, common, None)
            e_t = [jnp.exp(_dec(T[g])) for g in range(_G)]

            def p2(i, ss):
                out = []
                for g in range(_G):
                    v = b0[i, pl.ds(16 * g, 16)]
                    ev = _enc(v)
                    eq = ev == T[g]
                    keep = (ev > T[g]) | (eq & (ss[g] < TRUNC_R))
                    b0[i, pl.ds(16 * g, 16)] = jnp.where(keep, v, NEG_FILL)
                    out.append(ss[g] + jnp.where(eq, e_t[g], 0.0))
                return tuple(out)

            ss = A
            for c in range(_NCH):
                pltpu.sync_copy(_chunk(x_hbm, b, t0, c), b0)
                ss = lax.fori_loop(0, _CH, p2, ss)
                pltpu.sync_copy(b0, _chunk(out_hbm, b, t0, c))
            return 0

        lax.cond(n_slow > 0, slow, fast, None)
        return carry

    lax.fori_loop(0, _NSLAB, slab, 0)


_sc_call = functools.partial(
    pl.kernel,
    out_type=jax.ShapeDtypeStruct((_B, _V, _S), jnp.float32),
    mesh=plsc.VectorSubcoreMesh(core_axis_name="c", subcore_axis_name="s"),
    scratch_types=[
        pltpu.VMEM((_CH, _W), jnp.float32),
        pltpu.VMEM((_CH, _W), jnp.float32),
        pltpu.VMEM((_CH, _W), jnp.float32),
        pltpu.SemaphoreType.DMA,
        pltpu.SemaphoreType.DMA,
        pltpu.SemaphoreType.DMA,
        pltpu.SemaphoreType.DMA,
        pltpu.SemaphoreType.DMA,
        pltpu.SemaphoreType.DMA,
    ],
    compiler_params=pltpu.CompilerParams(needs_layout_passes=False),
)(_body)


@jax.jit
def kernel(logits):
    return _sc_call(logits)


# speedup vs baseline: 73.6185x; 1.0122x over previous
import functools

import jax
import jax.numpy as jnp
from jax import lax
from jax.experimental import pallas as pl
from jax.experimental.pallas import tpu as pltpu
from jax.experimental.pallas import tpu_sc as plsc

TRUNC_R = 0.85
NEG_FILL = -70.0

_B, _V, _S = 16, 1024, 2048
_L = 16
_W = 128
_G = _W // _L
_CH = 256
_NCH = _V // _CH

_info = plsc.get_sparse_core_info()
_NC, _NS = _info.num_cores, _info.num_subcores
_NW = _NC * _NS
_NSLAB = (_B * (_S // _W)) // _NW


def _enc(v):
    u = plsc.bitcast(v, jnp.uint32)
    s = u >> jnp.uint32(31)
    mask = s * jnp.uint32(0x7FFFFFFF) + jnp.uint32(0x80000000)
    return u ^ mask


def _dec(e):
    s = e >> jnp.uint32(31)
    mask = jnp.uint32(0xFFFFFFFF) - s * jnp.uint32(0x7FFFFFFF)
    return plsc.bitcast(e ^ mask, jnp.float32)


def _chunk(ref, b, t0, c):
    return ref.at[b, pl.ds(c * _CH, _CH), pl.ds(t0, _W)]


def _body(x_hbm, out_hbm, b0, b1, b2, is0, is1, is2, os0, os1, os2):
    wid = lax.axis_index("s") * _NC + lax.axis_index("c")
    bufs = (b0, b1, b2)
    isems = (is0, is1, is2)
    osems = (os0, os1, os2)

    def slab(j, carry):
        s = wid * _NSLAB + j
        b = s // (_S // _W)
        t0 = (s % (_S // _W)) * _W

        for c in range(3):
            pltpu.async_copy(_chunk(x_hbm, b, t0, c), bufs[c], isems[c])

        def p1(buf):
            def rows(i, mm):
                m1, m2 = mm
                base = i * 2
                for r in range(2):
                    vs = [buf[base + r, pl.ds(16 * g, 16)] for g in range(_G)]
                    mn = [jnp.minimum(m1[g], vs[g]) for g in range(_G)]
                    m1 = tuple(
                        jnp.maximum(m1[g], vs[g]) for g in range(_G)
                    )
                    m2 = tuple(jnp.maximum(m2[g], mn[g]) for g in range(_G))
                return m1, m2

            return rows

        ninf = tuple(jnp.full((_L,), -jnp.inf, jnp.float32) for _ in range(_G))
        mm = (ninf, ninf)
        for c in range(_NCH):
            pltpu.make_async_copy(
                _chunk(x_hbm, b, t0, c), bufs[c % 3], isems[c % 3]
            ).wait()
            if 3 <= c + 1 < _NCH:
                pltpu.async_copy(
                    _chunk(x_hbm, b, t0, c + 1), bufs[(c + 1) % 3],
                    isems[(c + 1) % 3],
                )
            mm = lax.fori_loop(0, _CH // 2, p1(bufs[c % 3]), mm)
        m1, m2 = mm

        em = [jnp.exp(m1[g]) for g in range(_G)]
        tie = [m2[g] == m1[g] for g in range(_G)]
        rare = [em[g] < TRUNC_R for g in range(_G)]
        n_slow = sum(
            plsc.all_reduce_population_count(tie[g] | rare[g])[0]
            for g in range(_G)
        )
        n_rare = sum(
            plsc.all_reduce_population_count(rare[g])[0] for g in range(_G)
        )

        def fast(_):
            def pf(buf):
                def rows(i, carry):
                    for g in range(_G):
                        v = buf[i, pl.ds(16 * g, 16)]
                        buf[i, pl.ds(16 * g, 16)] = jnp.where(
                            v == m1[g], v, NEG_FILL
                        )
                    return carry

                return rows

            lax.fori_loop(0, _CH, pf(b0), 0)
            pltpu.async_copy(b0, _chunk(out_hbm, b, t0, 3), os0)
            lax.fori_loop(0, _CH, pf(b2), 0)
            pltpu.async_copy(b2, _chunk(out_hbm, b, t0, 2), os2)
            pltpu.make_async_copy(b0, _chunk(out_hbm, b, t0, 3), os0).wait()
            pltpu.async_copy(_chunk(x_hbm, b, t0, 0), b0, is0)
            lax.fori_loop(0, _CH, pf(b1), 0)
            pltpu.async_copy(b1, _chunk(out_hbm, b, t0, 1), os1)
            pltpu.make_async_copy(_chunk(x_hbm, b, t0, 0), b0, is0).wait()
            lax.fori_loop(0, _CH, pf(b0), 0)
            pltpu.async_copy(b0, _chunk(out_hbm, b, t0, 0), os0)
            pltpu.make_async_copy(b0, _chunk(out_hbm, b, t0, 0), os0).wait()
            pltpu.make_async_copy(b1, _chunk(out_hbm, b, t0, 1), os1).wait()
            pltpu.make_async_copy(b2, _chunk(out_hbm, b, t0, 2), os2).wait()
            return 0

        def slow(_):
            def mass_gt(thr):
                def rows(i, acc):
                    return tuple(
                        acc[g]
                        + jnp.where(
                            _enc(b0[i, pl.ds(16 * g, 16)]) > thr[g],
                            jnp.exp(b0[i, pl.ds(16 * g, 16)]),
                            0.0,
                        )
                        for g in range(_G)
                    )

                acc = tuple(jnp.zeros((_L,), jnp.float32) for _ in range(_G))
                for c in range(_NCH):
                    pltpu.sync_copy(_chunk(x_hbm, b, t0, c), b0)
                    acc = lax.fori_loop(0, _CH, rows, acc)
                return acc

            def common(_):
                return (
                    tuple(_enc(m1[g]) for g in range(_G)),
                    tuple(jnp.zeros((_L,), jnp.float32) for _ in range(_G)),
                )

            def search(_):
                def sb(_i, c):
                    lo, hi = c
                    mid = tuple(
                        lo[g] + ((hi[g] - lo[g]) >> jnp.uint32(1))
                        for g in range(_G)
                    )
                    pred = [mg < TRUNC_R for mg in mass_gt(mid)]
                    return (
                        tuple(
                            jnp.where(pred[g], lo[g], mid[g] + jnp.uint32(1))
                            for g in range(_G)
                        ),
                        tuple(
                            jnp.where(pred[g], mid[g], hi[g])
                            for g in range(_G)
                        ),
                    )

                lo0 = tuple(jnp.zeros((_L,), jnp.uint32) for _ in range(_G))
                hi0 = tuple(
                    jnp.full((_L,), 0xFFFFFFFF, jnp.uint32) for _ in range(_G)
                )
                _, thr = lax.fori_loop(0, 32, sb, (lo0, hi0))
                return thr, mass_gt(thr)

            T, A = lax.cond(n_rare > 0, search, common, None)
            e_t = [jnp.exp(_dec(T[g])) for g in range(_G)]

            def p2(i, ss):
                out = []
                for g in range(_G):
                    v = b0[i, pl.ds(16 * g, 16)]
                    ev = _enc(v)
                    eq = ev == T[g]
                    keep = (ev > T[g]) | (eq & (ss[g] < TRUNC_R))
                    b0[i, pl.ds(16 * g, 16)] = jnp.where(keep, v, NEG_FILL)
                    out.append(ss[g] + jnp.where(eq, e_t[g], 0.0))
                return tuple(out)

            ss = A
            for c in range(_NCH):
                pltpu.sync_copy(_chunk(x_hbm, b, t0, c), b0)
                ss = lax.fori_loop(0, _CH, p2, ss)
                pltpu.sync_copy(b0, _chunk(out_hbm, b, t0, c))
            return 0

        lax.cond(n_slow > 0, slow, fast, None)
        return carry

    lax.fori_loop(0, _NSLAB, slab, 0)


_sc_call = functools.partial(
    pl.kernel,
    out_type=jax.ShapeDtypeStruct((_B, _V, _S), jnp.float32),
    mesh=plsc.VectorSubcoreMesh(core_axis_name="c", subcore_axis_name="s"),
    scratch_types=[
        pltpu.VMEM((_CH, _W), jnp.float32),
        pltpu.VMEM((_CH, _W), jnp.float32),
        pltpu.VMEM((_CH, _W), jnp.float32),
        pltpu.SemaphoreType.DMA,
        pltpu.SemaphoreType.DMA,
        pltpu.SemaphoreType.DMA,
        pltpu.SemaphoreType.DMA,
        pltpu.SemaphoreType.DMA,
        pltpu.SemaphoreType.DMA,
    ],
    compiler_params=pltpu.CompilerParams(needs_layout_passes=False),
)(_body)


@jax.jit
def kernel(logits):
    return _sc_call(logits)


# speedup vs baseline: 73.6991x; 1.0011x over previous
import functools

import jax
import jax.numpy as jnp
from jax import lax
from jax.experimental import pallas as pl
from jax.experimental.pallas import tpu as pltpu
from jax.experimental.pallas import tpu_sc as plsc

TRUNC_R = 0.85
NEG_FILL = -70.0

_B, _V, _S = 16, 1024, 2048
_L = 16
_W = 128
_G = _W // _L
_CH = 256
_NCH = _V // _CH

_info = plsc.get_sparse_core_info()
_NC, _NS = _info.num_cores, _info.num_subcores
_NW = _NC * _NS
_NSLAB = (_B * (_S // _W)) // _NW


def _enc(v):
    u = plsc.bitcast(v, jnp.uint32)
    s = u >> jnp.uint32(31)
    mask = s * jnp.uint32(0x7FFFFFFF) + jnp.uint32(0x80000000)
    return u ^ mask


def _dec(e):
    s = e >> jnp.uint32(31)
    mask = jnp.uint32(0xFFFFFFFF) - s * jnp.uint32(0x7FFFFFFF)
    return plsc.bitcast(e ^ mask, jnp.float32)


def _chunk(ref, b, t0, c):
    return ref.at[b, pl.ds(c * _CH, _CH), pl.ds(t0, _W)]


def _body(x_hbm, out_hbm, b0, b1, b2, is0, is1, is2, os0, os1, os2):
    wid = lax.axis_index("s") * _NC + lax.axis_index("c")
    bufs = (b0, b1, b2)
    isems = (is0, is1, is2)
    osems = (os0, os1, os2)

    def slab(j, carry):
        s = wid * _NSLAB + j
        b = s // (_S // _W)
        t0 = (s % (_S // _W)) * _W

        for c in range(3):
            pltpu.async_copy(_chunk(x_hbm, b, t0, c), bufs[c], isems[c])

        def p1(buf):
            def rows(i, mm):
                m1, m2 = mm
                base = i * 2
                for r in range(2):
                    vs = [buf[base + r, pl.ds(16 * g, 16)] for g in range(_G)]
                    mn = [jnp.minimum(m1[g], vs[g]) for g in range(_G)]
                    m1 = tuple(
                        jnp.maximum(m1[g], vs[g]) for g in range(_G)
                    )
                    m2 = tuple(jnp.maximum(m2[g], mn[g]) for g in range(_G))
                return m1, m2

            return rows

        ninf = tuple(jnp.full((_L,), -jnp.inf, jnp.float32) for _ in range(_G))
        mm = (ninf, ninf)
        for c in range(_NCH):
            pltpu.make_async_copy(
                _chunk(x_hbm, b, t0, c), bufs[c % 3], isems[c % 3]
            ).wait()
            if 3 <= c + 1 < _NCH:
                pltpu.async_copy(
                    _chunk(x_hbm, b, t0, c + 1), bufs[(c + 1) % 3],
                    isems[(c + 1) % 3],
                )
            mm = lax.fori_loop(0, _CH // 2, p1(bufs[c % 3]), mm)
        m1, m2 = mm

        em = [jnp.exp(m1[g]) for g in range(_G)]
        tie = [m2[g] == m1[g] for g in range(_G)]
        rare = [em[g] < TRUNC_R for g in range(_G)]
        n_slow = sum(
            plsc.all_reduce_population_count(tie[g] | rare[g])[0]
            for g in range(_G)
        )
        n_rare = sum(
            plsc.all_reduce_population_count(rare[g])[0] for g in range(_G)
        )

        def fast(_):
            def pf(buf):
                def rows(i, carry):
                    base = i * 2
                    for r in range(2):
                        for g in range(_G):
                            v = buf[base + r, pl.ds(16 * g, 16)]
                            buf[base + r, pl.ds(16 * g, 16)] = jnp.where(
                                v == m1[g], v, NEG_FILL
                            )
                    return carry

                return rows

            lax.fori_loop(0, _CH // 2, pf(b0), 0)
            pltpu.async_copy(b0, _chunk(out_hbm, b, t0, 3), os0)
            lax.fori_loop(0, _CH // 2, pf(b2), 0)
            pltpu.async_copy(b2, _chunk(out_hbm, b, t0, 2), os2)
            pltpu.make_async_copy(b0, _chunk(out_hbm, b, t0, 3), os0).wait()
            pltpu.async_copy(_chunk(x_hbm, b, t0, 0), b0, is0)
            lax.fori_loop(0, _CH // 2, pf(b1), 0)
            pltpu.async_copy(b1, _chunk(out_hbm, b, t0, 1), os1)
            pltpu.make_async_copy(_chunk(x_hbm, b, t0, 0), b0, is0).wait()
            lax.fori_loop(0, _CH // 2, pf(b0), 0)
            pltpu.async_copy(b0, _chunk(out_hbm, b, t0, 0), os0)
            pltpu.make_async_copy(b0, _chunk(out_hbm, b, t0, 0), os0).wait()
            pltpu.make_async_copy(b1, _chunk(out_hbm, b, t0, 1), os1).wait()
            pltpu.make_async_copy(b2, _chunk(out_hbm, b, t0, 2), os2).wait()
            return 0

        def slow(_):
            def mass_gt(thr):
                def rows(i, acc):
                    return tuple(
                        acc[g]
                        + jnp.where(
                            _enc(b0[i, pl.ds(16 * g, 16)]) > thr[g],
                            jnp.exp(b0[i, pl.ds(16 * g, 16)]),
                            0.0,
                        )
                        for g in range(_G)
                    )

                acc = tuple(jnp.zeros((_L,), jnp.float32) for _ in range(_G))
                for c in range(_NCH):
                    pltpu.sync_copy(_chunk(x_hbm, b, t0, c), b0)
                    acc = lax.fori_loop(0, _CH, rows, acc)
                return acc

            def common(_):
                return (
                    tuple(_enc(m1[g]) for g in range(_G)),
                    tuple(jnp.zeros((_L,), jnp.float32) for _ in range(_G)),
                )

            def search(_):
                def sb(_i, c):
                    lo, hi = c
                    mid = tuple(
                        lo[g] + ((hi[g] - lo[g]) >> jnp.uint32(1))
                        for g in range(_G)
                    )
                    pred = [mg < TRUNC_R for mg in mass_gt(mid)]
                    return (
                        tuple(
                            jnp.where(pred[g], lo[g], mid[g] + jnp.uint32(1))
                            for g in range(_G)
                        ),
                        tuple(
                            jnp.where(pred[g], mid[g], hi[g])
                            for g in range(_G)
                        ),
                    )

                lo0 = tuple(jnp.zeros((_L,), jnp.uint32) for _ in range(_G))
                hi0 = tuple(
                    jnp.full((_L,), 0xFFFFFFFF, jnp.uint32) for _ in range(_G)
                )
                _, thr = lax.fori_loop(0, 32, sb, (lo0, hi0))
                return thr, mass_gt(thr)

            T, A = lax.cond(n_rare > 0, search, common, None)
            e_t = [jnp.exp(_dec(T[g])) for g in range(_G)]

            def p2(i, ss):
                out = []
                for g in range(_G):
                    v = b0[i, pl.ds(16 * g, 16)]
                    ev = _enc(v)
                    eq = ev == T[g]
                    keep = (ev > T[g]) | (eq & (ss[g] < TRUNC_R))
                    b0[i, pl.ds(16 * g, 16)] = jnp.where(keep, v, NEG_FILL)
                    out.append(ss[g] + jnp.where(eq, e_t[g], 0.0))
                return tuple(out)

            ss = A
            for c in range(_NCH):
                pltpu.sync_copy(_chunk(x_hbm, b, t0, c), b0)
                ss = lax.fori_loop(0, _CH, p2, ss)
                pltpu.sync_copy(b0, _chunk(out_hbm, b, t0, c))
            return 0

        lax.cond(n_slow > 0, slow, fast, None)
        return carry

    lax.fori_loop(0, _NSLAB, slab, 0)


_sc_call = functools.partial(
    pl.kernel,
    out_type=jax.ShapeDtypeStruct((_B, _V, _S), jnp.float32),
    mesh=plsc.VectorSubcoreMesh(core_axis_name="c", subcore_axis_name="s"),
    scratch_types=[
        pltpu.VMEM((_CH, _W), jnp.float32),
        pltpu.VMEM((_CH, _W), jnp.float32),
        pltpu.VMEM((_CH, _W), jnp.float32),
        pltpu.SemaphoreType.DMA,
        pltpu.SemaphoreType.DMA,
        pltpu.SemaphoreType.DMA,
        pltpu.SemaphoreType.DMA,
        pltpu.SemaphoreType.DMA,
        pltpu.SemaphoreType.DMA,
    ],
    compiler_params=pltpu.CompilerParams(needs_layout_passes=False),
)(_body)


@jax.jit
def kernel(logits):
    return _sc_call(logits)
